# baseline (speedup 1.0000x reference)
import jax
import jax.numpy as jnp
from jax import lax
from jax.experimental import pallas as pl
from jax.experimental.pallas import tpu as pltpu
from jax.experimental.pallas import tpu_sc as plsc

_B = 8
_N = 5000
_C = 81
_D = 256
_IMG = 512.0
_CENTER_VAR = 0.1
_SIZE_VAR = 0.2
_NMS_THRESH = 0.45
_MAX_CAND = 100
_NEG = -1e30
_NPAD = 5120
_NKEYMIN = -2147483648


def _stage1_body(feat_ref, wc_ref, bc_ref, wb_ref, bb_ref, pri_ref,
                 scores_ref, amax_ref):
  feat = feat_ref[0]
  logits = jnp.dot(feat, wc_ref[...], preferred_element_type=jnp.float32)
  logits = logits + bc_ref[...]
  m = jnp.max(logits, axis=1, keepdims=True)
  p = logits - m
  lse = jnp.log(jnp.sum(jnp.exp(p), axis=1, keepdims=True))
  logp = p - lse
  lane = lax.broadcasted_iota(jnp.int32, (_N, 128), 1)
  valid = jnp.logical_and(lane >= 1, lane <= _C - 1)
  scores = jnp.where(valid, logp, _NEG)
  amax_ref[0] = jnp.max(scores, axis=1, keepdims=True)

  loc = jnp.dot(feat, wb_ref[...], preferred_element_type=jnp.float32)
  loc = loc + bb_ref[...]
  pri = pri_ref[...]
  pw = pri[:, 2:3]
  ph = pri[:, 3:4]
  cx = loc[:, 0:1] * _CENTER_VAR * pw + pri[:, 0:1]
  cy = loc[:, 1:2] * _CENTER_VAR * ph + pri[:, 1:2]
  w = jnp.exp(loc[:, 2:3] * _SIZE_VAR) * pw
  h = jnp.exp(loc[:, 3:4] * _SIZE_VAR) * ph
  boxmat = jnp.concatenate(
      [jnp.zeros((_N, 96), jnp.float32),
       cx - w * 0.5, cy - h * 0.5, cx + w * 0.5, cy + h * 0.5,
       jnp.zeros((_N, 28), jnp.float32)], axis=1)
  emb = jnp.logical_and(lane >= 96, lane < 100)
  scores_ref[0] = jnp.where(emb, boxmat, scores)


def _scalar(x):
  return x[0] if getattr(x, "ndim", 0) else x


_GDN = lax.GatherDimensionNumbers(
    offset_dims=(), collapsed_slice_dims=(0,), start_index_map=(0,))


def _shuf(v, idx):
  return lax.gather(v, idx[:, None], dimension_numbers=_GDN,
                    slice_sizes=(1,), mode=lax.GatherScatterMode.PROMISE_IN_BOUNDS)


def _vmax(v, lane16):
  for s in (8, 4, 2, 1):
    v = jnp.maximum(v, _shuf(v, lane16 ^ s))
  return v[0]


def _lanebcast(v, lane16, i):
  return _shuf(v, lane16 * 0 + i)


def _ffs(mask, lane16):
  c = jnp.where(mask, lane16, jnp.int32(1 << 20))
  for s in (8, 4, 2, 1):
    c = jnp.minimum(c, _shuf(c, lane16 ^ s))
  return c[0]


def _popcnt(mask, lane16):
  c = jnp.where(mask, jnp.int32(1), jnp.int32(0))
  for s in (8, 4, 2, 1):
    c = c + _shuf(c, lane16 ^ s)
  return c[0]


def _sc_body(amax_hbm, scores_hbm,
             boxes_o, labels_o, scores_o,
             s_amax, s_l1, s_l2, s_sorted, s_rows,
             s_p1, s_p2, cnd_s, cnd_c,
             rx1, ry1, rx2, ry2, cx1, cy1, cx2, cy2, s_area, s_supp,
             s_ob, sem):
  nc = 2
  wid = lax.axis_index("s") * nc + lax.axis_index("c")

  @pl.when(wid < _B)
  def _():
    img = wid
    base = img * _N
    lane16 = lax.iota(jnp.int32, 16)
    negv = jnp.full((16,), _NEG, jnp.float32)

    pltpu.sync_copy(amax_hbm.at[img], s_amax)

    def build_l1(o, _):
      acc = negv
      for t in range(16):
        val = _vmax(s_amax[pl.ds(o * 256 + t * 16, 16)], lane16)
        acc = jnp.where(lane16 == t, val, acc)
      s_l1[pl.ds(o * 16, 16)] = acc
      return 0
    lax.fori_loop(0, 20, build_l1, 0)

    for o in range(2):
      acc = negv
      for t in range(16 if o == 0 else 4):
        val = _vmax(s_l1[pl.ds((o * 16 + t) * 16, 16)], lane16)
        acc = jnp.where(lane16 == t, val, acc)
      s_l2[pl.ds(o * 16, 16)] = acc

    def extract_a(k, _):
      va = s_l2[pl.ds(0, 16)]
      vb = s_l2[pl.ds(16, 16)]
      ma = _vmax(va, lane16)
      mb = _vmax(vb, lane16)
      in_a = ma >= mb
      v = jnp.where(in_a, ma, mb)
      vec = jnp.where(in_a, va, vb)
      bse = jnp.where(in_a, 0, 16)
      j2 = bse + _ffs(vec == v, lane16)
      lv = s_l1[pl.ds(j2 * 16, 16)]
      j1 = j2 * 16 + _ffs(lv == v, lane16)
      av = s_amax[pl.ds(j1 * 16, 16)]
      alane = _ffs(av == v, lane16)
      s_amax[pl.ds(j1 * 16, 16)] = jnp.where(lane16 == alane, _NEG, av)
      nv1 = _vmax(s_amax[pl.ds(j1 * 16, 16)], lane16)
      o1 = (j1 // 16) * 16
      s_l1[pl.ds(o1, 16)] = jnp.where(lane16 == j1 % 16, nv1,
                                      s_l1[pl.ds(o1, 16)])
      nv2 = _vmax(s_l1[pl.ds(j2 * 16, 16)], lane16)
      o2 = (j2 // 16) * 16
      s_l2[pl.ds(o2, 16)] = jnp.where(lane16 == j2 % 16, nv2,
                                      s_l2[pl.ds(o2, 16)])
      return 0
    lax.fori_loop(0, _MAX_CAND, extract_a, 0)

    def init_sorted(j, _):
      s_sorted[pl.ds(j * 16, 16)] = lane16 + (base + j * 16)
      return 0
    lax.fori_loop(0, 8, init_sorted, 0)

    def compact(j, cnt):
      av = s_amax[pl.ds(j * 16, 16)]
      flat = lane16 + j * 16
      msk = jnp.logical_and(av == _NEG, flat < _N)
      mvec = jnp.where(msk, jnp.int32(1), jnp.int32(0))
      n = _popcnt(msk, lane16)

      def inner(_, st):
        mcur, c = st
        lanei = _ffs(mcur == 1, lane16)
        anchor = j * 16 + lanei + base
        co = (c // 16) * 16
        s_sorted[pl.ds(co, 16)] = jnp.where(lane16 == c % 16, anchor,
                                            s_sorted[pl.ds(co, 16)])
        mnext = jnp.where(lane16 == lanei, jnp.int32(0), mcur)
        return (mnext, c + 1)

      st = lax.fori_loop(0, n, inner, (mvec, cnt))
      return st[1]
    lax.fori_loop(0, 313, compact, jnp.int32(0))

    pltpu.async_copy(scores_hbm.at[s_sorted], s_rows, sem).wait()

    def build_p1(o, _):
      acc = negv
      for t in range(16):
        r = o * 2 + t // 8
        c0 = (t % 8) * 16
        x = s_rows[r, pl.ds(c0, 16)]
        if t % 8 == 6:
          val = _NEG
        else:
          val = jnp.where(r < _MAX_CAND, _vmax(x, lane16), _NEG)
        acc = jnp.where(lane16 == t, val, acc)
      s_p1[pl.ds(o * 16, 16)] = acc
      return 0
    lax.fori_loop(0, 64, build_p1, 0)

    def build_p2(o, _):
      acc = negv
      for t in range(16):
        val = _vmax(s_p1[pl.ds((o * 16 + t) * 16, 16)], lane16)
        acc = jnp.where(lane16 == t, val, acc)
      s_p2[pl.ds(o * 16, 16)] = acc
      return 0
    lax.fori_loop(0, 4, build_p2, 0)

    def init_cand(j, _):
      z = jnp.zeros((16,), jnp.float32)
      cnd_s[pl.ds(j * 16, 16)] = z
      cnd_c[pl.ds(j * 16, 16)] = jnp.zeros((16,), jnp.int32)
      rx1[pl.ds(j * 16, 16)] = z
      ry1[pl.ds(j * 16, 16)] = z
      rx2[pl.ds(j * 16, 16)] = z
      ry2[pl.ds(j * 16, 16)] = z
      cx1[pl.ds(j * 16, 16)] = z
      cy1[pl.ds(j * 16, 16)] = z
      cx2[pl.ds(j * 16, 16)] = z
      cy2[pl.ds(j * 16, 16)] = z
      s_supp[pl.ds(j * 16, 16)] = z
      return 0
    lax.fori_loop(0, 8, init_cand, 0)

    def extract_p(k, _):
      p20 = s_p2[pl.ds(0, 16)]
      p21 = s_p2[pl.ds(16, 16)]
      p22 = s_p2[pl.ds(32, 16)]
      p23 = s_p2[pl.ds(48, 16)]
      m0 = _vmax(p20, lane16)
      m1 = _vmax(p21, lane16)
      m2 = _vmax(p22, lane16)
      m3 = _vmax(p23, lane16)
      v = jnp.maximum(jnp.maximum(m0, m1), jnp.maximum(m2, m3))
      in0 = m0 == v
      in1 = m1 == v
      in2 = m2 == v
      bse = jnp.where(in0, 0, jnp.where(in1, 16, jnp.where(in2, 32, 48)))
      vec = jnp.where(in0, p20, jnp.where(in1, p21, jnp.where(in2, p22, p23)))
      t = bse + _ffs(vec == v, lane16)
      pv = s_p1[pl.ds(t * 16, 16)]
      j = t * 16 + _ffs(pv == v, lane16)
      r = j // 8
      c0 = (j % 8) * 16
      leaf = s_rows[r, pl.ds(c0, 16)]
      lne = _ffs(leaf == v, lane16)
      c = c0 + lne
      ko = (k // 16) * 16
      ksl = pl.ds(ko, 16)
      klane = lane16 == k % 16
      sc = v
      cnd_s[ksl] = jnp.where(klane, sc, cnd_s[ksl])
      cnd_c[ksl] = jnp.where(klane, c, cnd_c[ksl])
      bv = s_rows[r, pl.ds(96, 16)]
      x1 = bv[0] * _IMG
      y1 = bv[1] * _IMG
      x2 = bv[2] * _IMG
      y2 = bv[3] * _IMG
      off = c.astype(jnp.float32) * (_IMG * 4.0)
      rx1[ksl] = jnp.where(klane, x1, rx1[ksl])
      ry1[ksl] = jnp.where(klane, y1, ry1[ksl])
      rx2[ksl] = jnp.where(klane, x2, rx2[ksl])
      ry2[ksl] = jnp.where(klane, y2, ry2[ksl])
      cx1[ksl] = jnp.where(klane, x1 + off, cx1[ksl])
      cy1[ksl] = jnp.where(klane, y1 + off, cy1[ksl])
      cx2[ksl] = jnp.where(klane, x2 + off, cx2[ksl])
      cy2[ksl] = jnp.where(klane, y2 + off, cy2[ksl])
      s_rows[r, pl.ds(c0, 16)] = jnp.where(lane16 == lne, _NEG, leaf)
      nv1 = _vmax(s_rows[r, pl.ds(c0, 16)], lane16)
      o1 = (j // 16) * 16
      s_p1[pl.ds(o1, 16)] = jnp.where(lane16 == j % 16, nv1,
                                      s_p1[pl.ds(o1, 16)])
      t2 = j // 16
      nv2 = _vmax(s_p1[pl.ds(t2 * 16, 16)], lane16)
      o2 = (t2 // 16) * 16
      s_p2[pl.ds(o2, 16)] = jnp.where(lane16 == t2 % 16, nv2,
                                      s_p2[pl.ds(o2, 16)])
      return 0
    lax.fori_loop(0, _MAX_CAND, extract_p, 0)

    def build_area(j, _):
      w = jnp.maximum(cx2[pl.ds(j * 16, 16)] - cx1[pl.ds(j * 16, 16)], 0.0)
      h = jnp.maximum(cy2[pl.ds(j * 16, 16)] - cy1[pl.ds(j * 16, 16)], 0.0)
      s_area[pl.ds(j * 16, 16)] = w * h
      return 0
    lax.fori_loop(0, 8, build_area, 0)

    def nms_step(i, _):
      io = (i // 16) * 16
      isl = pl.ds(io, 16)
      ilane = lane16 == i % 16

      def pick(ref):
        return _lanebcast(ref[isl], lane16, i % 16)

      live = jnp.where(pick(s_supp) == 0.0, 1.0, 0.0)
      x1i = pick(cx1)
      y1i = pick(cy1)
      x2i = pick(cx2)
      y2i = pick(cy2)
      ai = pick(s_area)
      for j in range(8):
        sl = pl.ds(j * 16, 16)
        ltx = jnp.maximum(x1i, cx1[sl])
        lty = jnp.maximum(y1i, cy1[sl])
        rbx = jnp.minimum(x2i, cx2[sl])
        rby = jnp.minimum(y2i, cy2[sl])
        inter = jnp.maximum(rbx - ltx, 0.0) * jnp.maximum(rby - lty, 0.0)
        union = ai + s_area[sl] - inter
        iou = inter / jnp.maximum(union, 1e-9)
        glane = lane16 + j * 16
        hit = jnp.logical_and(iou > _NMS_THRESH, glane > i)
        s_supp[sl] = jnp.maximum(s_supp[sl],
                                 jnp.where(hit, live, 0.0))
      return 0
    lax.fori_loop(0, _MAX_CAND, nms_step, 0)

    def mask_vec(j, _):
      sl = pl.ds(j * 16, 16)
      kp = s_supp[sl] == 0.0
      f = jnp.where(kp, 1.0, 0.0)
      cnd_s[sl] = jnp.where(kp, cnd_s[sl], -1e4)
      cnd_c[sl] = jnp.where(kp, cnd_c[sl], -1)
      s_ob[0, sl] = rx1[sl] * f
      s_ob[1, sl] = ry1[sl] * f
      s_ob[2, sl] = rx2[sl] * f
      s_ob[3, sl] = ry2[sl] * f
      return 0
    lax.fori_loop(0, 8, mask_vec, 0)

    pltpu.sync_copy(s_ob, boxes_o.at[img])
    pltpu.sync_copy(cnd_c, labels_o.at[img])
    pltpu.sync_copy(cnd_s, scores_o.at[img])


@jax.jit
def kernel(features, W_cls, b_cls, W_box, b_box, priors):
  wc = jnp.zeros((_D, 128), jnp.float32).at[:, :_C].set(W_cls)
  bc = jnp.full((1, 128), _NEG, jnp.float32).at[0, :_C].set(b_cls)
  bb = b_box.reshape(1, 4)

  scores, amax = pl.pallas_call(
      _stage1_body,
      grid=(_B,),
      in_specs=[
          pl.BlockSpec((1, _N, _D), lambda i: (i, 0, 0)),
          pl.BlockSpec((_D, 128), lambda i: (0, 0)),
          pl.BlockSpec((1, 128), lambda i: (0, 0)),
          pl.BlockSpec((_D, 4), lambda i: (0, 0)),
          pl.BlockSpec((1, 4), lambda i: (0, 0)),
          pl.BlockSpec((_N, 4), lambda i: (0, 0)),
      ],
      out_specs=[
          pl.BlockSpec((1, _N, 128), lambda i: (i, 0, 0)),
          pl.BlockSpec((1, _N, 1), lambda i: (i, 0, 0)),
      ],
      out_shape=[
          jax.ShapeDtypeStruct((_B, _N, 128), jnp.float32),
          jax.ShapeDtypeStruct((_B, _N, 1), jnp.float32),
      ],
  )(features, wc, bc, W_box, bb, priors)

  amax_pad = jnp.full((_B, _NPAD), _NEG, jnp.float32)
  amax_pad = amax_pad.at[:, :_N].set(amax[:, :, 0])
  scores_flat = scores.reshape(_B * _N, 128)

  mesh = plsc.VectorSubcoreMesh(core_axis_name="c", subcore_axis_name="s")
  sc_call = pl.kernel(
      _sc_body,
      mesh=mesh,
      out_type=[
          jax.ShapeDtypeStruct((_B, 4, 128), jnp.float32),
          jax.ShapeDtypeStruct((_B, 128), jnp.int32),
          jax.ShapeDtypeStruct((_B, 128), jnp.float32),
      ],
      scratch_types=[
          pltpu.VMEM((_NPAD,), jnp.float32),
          pltpu.VMEM((320,), jnp.float32),
          pltpu.VMEM((32,), jnp.float32),
          pltpu.VMEM((128,), jnp.int32),
          pltpu.VMEM((128, 128), jnp.float32),
          pltpu.VMEM((1024,), jnp.float32),
          pltpu.VMEM((64,), jnp.float32),
          pltpu.VMEM((128,), jnp.float32),
          pltpu.VMEM((128,), jnp.int32),
      ] + [pltpu.VMEM((128,), jnp.float32) for _ in range(10)] + [
          pltpu.VMEM((4, 128), jnp.float32),
          pltpu.SemaphoreType.DMA,
      ],
  )
  boxes_o, labels_o, scores_o = sc_call(amax_pad, scores_flat)

  boxes_out = jnp.transpose(boxes_o, (0, 2, 1))[:, :_MAX_CAND, :]
  labels_out = labels_o[:, :_MAX_CAND]
  scores_out = scores_o[:, :_MAX_CAND]
  img_w = jnp.full((_B,), _IMG, dtype=jnp.float32)
  img_h = jnp.full((_B,), _IMG, dtype=jnp.float32)
  return boxes_out, labels_out, scores_out, img_w, img_h


# speedup vs baseline: 1.0025x; 1.0025x over previous
import jax
import jax.numpy as jnp
from jax import lax
from jax.experimental import pallas as pl
from jax.experimental.pallas import tpu as pltpu
from jax.experimental.pallas import tpu_sc as plsc

_B = 8
_N = 5000
_C = 81
_D = 256
_IMG = 512.0
_CENTER_VAR = 0.1
_SIZE_VAR = 0.2
_NMS_THRESH = 0.45
_MAX_CAND = 100
_NEG = -1e30
_NPAD = 5120


def _stage1_body(feat_ref, wc_ref, bc_ref, wb_ref, bb_ref, pri_ref,
                 scores_ref, amax_ref):
  feat = feat_ref[0]
  logits = jnp.dot(feat, wc_ref[...], preferred_element_type=jnp.float32)
  logits = logits + bc_ref[...]
  m = jnp.max(logits, axis=1, keepdims=True)
  p = logits - m
  lse = jnp.log(jnp.sum(jnp.exp(p), axis=1, keepdims=True))
  logp = p - lse
  lane = lax.broadcasted_iota(jnp.int32, (_N, 128), 1)
  valid = jnp.logical_and(lane >= 1, lane <= _C - 1)
  scores = jnp.where(valid, logp, _NEG)
  amax_ref[0] = jnp.max(scores, axis=1, keepdims=True)

  loc = jnp.dot(feat, wb_ref[...], preferred_element_type=jnp.float32)
  loc = loc + bb_ref[...]
  pri = pri_ref[...]
  pw = pri[:, 2:3]
  ph = pri[:, 3:4]
  cx = loc[:, 0:1] * _CENTER_VAR * pw + pri[:, 0:1]
  cy = loc[:, 1:2] * _CENTER_VAR * ph + pri[:, 1:2]
  w = jnp.exp(loc[:, 2:3] * _SIZE_VAR) * pw
  h = jnp.exp(loc[:, 3:4] * _SIZE_VAR) * ph
  boxmat = jnp.concatenate(
      [jnp.zeros((_N, 96), jnp.float32),
       cx - w * 0.5, cy - h * 0.5, cx + w * 0.5, cy + h * 0.5,
       jnp.zeros((_N, 28), jnp.float32)], axis=1)
  emb = jnp.logical_and(lane >= 96, lane < 100)
  scores_ref[0] = jnp.where(emb, boxmat, scores)


_GDN = lax.GatherDimensionNumbers(
    offset_dims=(), collapsed_slice_dims=(0,), start_index_map=(0,))


def _shuf(v, idx):
  return lax.gather(v, idx[:, None], dimension_numbers=_GDN,
                    slice_sizes=(1,), mode=lax.GatherScatterMode.PROMISE_IN_BOUNDS)


def _vmax(v, lane16):
  for s in (8, 4, 2, 1):
    v = jnp.maximum(v, _shuf(v, lane16 ^ s))
  return v[0]


def _lanebcast(v, lane16, i):
  return _shuf(v, lane16 * 0 + i)


def _ffs(mask, lane16):
  c = jnp.where(mask, lane16, jnp.int32(1 << 20))
  for s in (8, 4, 2, 1):
    c = jnp.minimum(c, _shuf(c, lane16 ^ s))
  return c[0]


def _popcnt(mask, lane16):
  c = jnp.where(mask, jnp.int32(1), jnp.int32(0))
  for s in (8, 4, 2, 1):
    c = c + _shuf(c, lane16 ^ s)
  return c[0]


def _sc_body(amax_hbm, scores_hbm,
             boxes_o, labels_o, scores_o,
             s_amax, s_l1, s_l2, s_sorted, s_rows,
             s_p1, s_p2, cnd_s, cnd_c,
             rx1, ry1, rx2, ry2, cx1, cy1, cx2, cy2, s_area, s_supp,
             s_ob, sem):
  cidx = lax.axis_index("c")
  sidx = lax.axis_index("s")

  @pl.when(jnp.logical_and(cidx == 0, sidx < _B))
  def _():
    img = sidx
    base = img * _N
    lane16 = lax.iota(jnp.int32, 16)
    negv = jnp.full((16,), _NEG, jnp.float32)

    pltpu.sync_copy(amax_hbm.at[img], s_amax)

    def build_l1(o, _):
      acc = negv
      for t in range(16):
        val = _vmax(s_amax[pl.ds(o * 256 + t * 16, 16)], lane16)
        acc = jnp.where(lane16 == t, val, acc)
      s_l1[pl.ds(o * 16, 16)] = acc
      return 0
    lax.fori_loop(0, 20, build_l1, 0)

    for o in range(2):
      acc = negv
      for t in range(16 if o == 0 else 4):
        val = _vmax(s_l1[pl.ds((o * 16 + t) * 16, 16)], lane16)
        acc = jnp.where(lane16 == t, val, acc)
      s_l2[pl.ds(o * 16, 16)] = acc

    def extract_a(k, _):
      va = s_l2[pl.ds(0, 16)]
      vb = s_l2[pl.ds(16, 16)]
      ma = _vmax(va, lane16)
      mb = _vmax(vb, lane16)
      in_a = ma >= mb
      v = jnp.where(in_a, ma, mb)
      vec = jnp.where(in_a, va, vb)
      bse = jnp.where(in_a, 0, 16)
      j2 = bse + _ffs(vec == v, lane16)
      lv = s_l1[pl.ds(j2 * 16, 16)]
      j1 = j2 * 16 + _ffs(lv == v, lane16)
      av = s_amax[pl.ds(j1 * 16, 16)]
      alane = _ffs(av == v, lane16)
      s_amax[pl.ds(j1 * 16, 16)] = jnp.where(lane16 == alane, _NEG, av)
      nv1 = _vmax(s_amax[pl.ds(j1 * 16, 16)], lane16)
      o1 = (j1 // 16) * 16
      s_l1[pl.ds(o1, 16)] = jnp.where(lane16 == j1 % 16, nv1,
                                      s_l1[pl.ds(o1, 16)])
      nv2 = _vmax(s_l1[pl.ds(j2 * 16, 16)], lane16)
      o2 = (j2 // 16) * 16
      s_l2[pl.ds(o2, 16)] = jnp.where(lane16 == j2 % 16, nv2,
                                      s_l2[pl.ds(o2, 16)])
      return 0
    lax.fori_loop(0, _MAX_CAND, extract_a, 0)

    def init_sorted(j, _):
      s_sorted[pl.ds(j * 16, 16)] = lane16 + (base + j * 16)
      return 0
    lax.fori_loop(0, 8, init_sorted, 0)

    def compact(j, cnt):
      av = s_amax[pl.ds(j * 16, 16)]
      flat = lane16 + j * 16
      msk = jnp.logical_and(av == _NEG, flat < _N)
      mvec = jnp.where(msk, jnp.int32(1), jnp.int32(0))
      n = _popcnt(msk, lane16)

      def inner(_, st):
        mcur, c = st
        lanei = _ffs(mcur == 1, lane16)
        anchor = j * 16 + lanei + base
        co = (c // 16) * 16
        s_sorted[pl.ds(co, 16)] = jnp.where(lane16 == c % 16, anchor,
                                            s_sorted[pl.ds(co, 16)])
        mnext = jnp.where(lane16 == lanei, jnp.int32(0), mcur)
        return (mnext, c + 1)

      st = lax.fori_loop(0, n, inner, (mvec, cnt))
      return st[1]
    lax.fori_loop(0, 313, compact, jnp.int32(0))

    pltpu.async_copy(scores_hbm.at[s_sorted], s_rows, sem).wait()

    def build_p1(o, _):
      acc = negv
      for t in range(16):
        r = o * 2 + t // 8
        c0 = (t % 8) * 16
        x = s_rows[r, pl.ds(c0, 16)]
        if t % 8 == 6:
          val = _NEG
        else:
          val = jnp.where(r < _MAX_CAND, _vmax(x, lane16), _NEG)
        acc = jnp.where(lane16 == t, val, acc)
      s_p1[pl.ds(o * 16, 16)] = acc
      return 0
    lax.fori_loop(0, 64, build_p1, 0)

    def build_p2(o, _):
      acc = negv
      for t in range(16):
        val = _vmax(s_p1[pl.ds((o * 16 + t) * 16, 16)], lane16)
        acc = jnp.where(lane16 == t, val, acc)
      s_p2[pl.ds(o * 16, 16)] = acc
      return 0
    lax.fori_loop(0, 4, build_p2, 0)

    def init_cand(j, _):
      z = jnp.zeros((16,), jnp.float32)
      cnd_s[pl.ds(j * 16, 16)] = z
      cnd_c[pl.ds(j * 16, 16)] = jnp.zeros((16,), jnp.int32)
      rx1[pl.ds(j * 16, 16)] = z
      ry1[pl.ds(j * 16, 16)] = z
      rx2[pl.ds(j * 16, 16)] = z
      ry2[pl.ds(j * 16, 16)] = z
      cx1[pl.ds(j * 16, 16)] = z
      cy1[pl.ds(j * 16, 16)] = z
      cx2[pl.ds(j * 16, 16)] = z
      cy2[pl.ds(j * 16, 16)] = z
      s_supp[pl.ds(j * 16, 16)] = z
      return 0
    lax.fori_loop(0, 8, init_cand, 0)

    def extract_p(k, _):
      p20 = s_p2[pl.ds(0, 16)]
      p21 = s_p2[pl.ds(16, 16)]
      p22 = s_p2[pl.ds(32, 16)]
      p23 = s_p2[pl.ds(48, 16)]
      m0 = _vmax(p20, lane16)
      m1 = _vmax(p21, lane16)
      m2 = _vmax(p22, lane16)
      m3 = _vmax(p23, lane16)
      v = jnp.maximum(jnp.maximum(m0, m1), jnp.maximum(m2, m3))
      in0 = m0 == v
      in1 = m1 == v
      in2 = m2 == v
      bse = jnp.where(in0, 0, jnp.where(in1, 16, jnp.where(in2, 32, 48)))
      vec = jnp.where(in0, p20, jnp.where(in1, p21, jnp.where(in2, p22, p23)))
      t = bse + _ffs(vec == v, lane16)
      pv = s_p1[pl.ds(t * 16, 16)]
      j = t * 16 + _ffs(pv == v, lane16)
      r = j // 8
      c0 = (j % 8) * 16
      leaf = s_rows[r, pl.ds(c0, 16)]
      lne = _ffs(leaf == v, lane16)
      c = c0 + lne
      ko = (k // 16) * 16
      ksl = pl.ds(ko, 16)
      klane = lane16 == k % 16
      sc = v
      cnd_s[ksl] = jnp.where(klane, sc, cnd_s[ksl])
      cnd_c[ksl] = jnp.where(klane, c, cnd_c[ksl])
      bv = s_rows[r, pl.ds(96, 16)]
      x1 = bv[0] * _IMG
      y1 = bv[1] * _IMG
      x2 = bv[2] * _IMG
      y2 = bv[3] * _IMG
      off = c.astype(jnp.float32) * (_IMG * 4.0)
      rx1[ksl] = jnp.where(klane, x1, rx1[ksl])
      ry1[ksl] = jnp.where(klane, y1, ry1[ksl])
      rx2[ksl] = jnp.where(klane, x2, rx2[ksl])
      ry2[ksl] = jnp.where(klane, y2, ry2[ksl])
      cx1[ksl] = jnp.where(klane, x1 + off, cx1[ksl])
      cy1[ksl] = jnp.where(klane, y1 + off, cy1[ksl])
      cx2[ksl] = jnp.where(klane, x2 + off, cx2[ksl])
      cy2[ksl] = jnp.where(klane, y2 + off, cy2[ksl])
      s_rows[r, pl.ds(c0, 16)] = jnp.where(lane16 == lne, _NEG, leaf)
      nv1 = _vmax(s_rows[r, pl.ds(c0, 16)], lane16)
      o1 = (j // 16) * 16
      s_p1[pl.ds(o1, 16)] = jnp.where(lane16 == j % 16, nv1,
                                      s_p1[pl.ds(o1, 16)])
      t2 = j // 16
      nv2 = _vmax(s_p1[pl.ds(t2 * 16, 16)], lane16)
      o2 = (t2 // 16) * 16
      s_p2[pl.ds(o2, 16)] = jnp.where(lane16 == t2 % 16, nv2,
                                      s_p2[pl.ds(o2, 16)])
      return 0
    lax.fori_loop(0, _MAX_CAND, extract_p, 0)

    def build_area(j, _):
      w = jnp.maximum(cx2[pl.ds(j * 16, 16)] - cx1[pl.ds(j * 16, 16)], 0.0)
      h = jnp.maximum(cy2[pl.ds(j * 16, 16)] - cy1[pl.ds(j * 16, 16)], 0.0)
      s_area[pl.ds(j * 16, 16)] = w * h
      return 0
    lax.fori_loop(0, 8, build_area, 0)

    def nms_step(i, _):
      io = (i // 16) * 16
      isl = pl.ds(io, 16)
      ilane = lane16 == i % 16

      def pick(ref):
        return _lanebcast(ref[isl], lane16, i % 16)

      live = jnp.where(pick(s_supp) == 0.0, 1.0, 0.0)
      x1i = pick(cx1)
      y1i = pick(cy1)
      x2i = pick(cx2)
      y2i = pick(cy2)
      ai = pick(s_area)
      for j in range(8):
        sl = pl.ds(j * 16, 16)
        ltx = jnp.maximum(x1i, cx1[sl])
        lty = jnp.maximum(y1i, cy1[sl])
        rbx = jnp.minimum(x2i, cx2[sl])
        rby = jnp.minimum(y2i, cy2[sl])
        inter = jnp.maximum(rbx - ltx, 0.0) * jnp.maximum(rby - lty, 0.0)
        union = ai + s_area[sl] - inter
        iou = inter / jnp.maximum(union, 1e-9)
        glane = lane16 + j * 16
        hit = jnp.logical_and(iou > _NMS_THRESH, glane > i)
        s_supp[sl] = jnp.maximum(s_supp[sl],
                                 jnp.where(hit, live, 0.0))
      return 0
    lax.fori_loop(0, _MAX_CAND, nms_step, 0)

    def mask_vec(j, _):
      sl = pl.ds(j * 16, 16)
      kp = s_supp[sl] == 0.0
      f = jnp.where(kp, 1.0, 0.0)
      cnd_s[sl] = jnp.where(kp, cnd_s[sl], -1e4)
      cnd_c[sl] = jnp.where(kp, cnd_c[sl], -1)
      s_ob[0, sl] = rx1[sl] * f
      s_ob[1, sl] = ry1[sl] * f
      s_ob[2, sl] = rx2[sl] * f
      s_ob[3, sl] = ry2[sl] * f
      return 0
    lax.fori_loop(0, 8, mask_vec, 0)

    pltpu.sync_copy(s_ob, boxes_o.at[img])
    pltpu.sync_copy(cnd_c, labels_o.at[img])
    pltpu.sync_copy(cnd_s, scores_o.at[img])


@jax.jit
def kernel(features, W_cls, b_cls, W_box, b_box, priors):
  wc = jnp.zeros((_D, 128), jnp.float32).at[:, :_C].set(W_cls)
  bc = jnp.full((1, 128), _NEG, jnp.float32).at[0, :_C].set(b_cls)
  bb = b_box.reshape(1, 4)

  scores, amax = pl.pallas_call(
      _stage1_body,
      grid=(_B,),
      in_specs=[
          pl.BlockSpec((1, _N, _D), lambda i: (i, 0, 0)),
          pl.BlockSpec((_D, 128), lambda i: (0, 0)),
          pl.BlockSpec((1, 128), lambda i: (0, 0)),
          pl.BlockSpec((_D, 4), lambda i: (0, 0)),
          pl.BlockSpec((1, 4), lambda i: (0, 0)),
          pl.BlockSpec((_N, 4), lambda i: (0, 0)),
      ],
      out_specs=[
          pl.BlockSpec((1, _N, 128), lambda i: (i, 0, 0)),
          pl.BlockSpec((1, _N, 1), lambda i: (i, 0, 0)),
      ],
      out_shape=[
          jax.ShapeDtypeStruct((_B, _N, 128), jnp.float32),
          jax.ShapeDtypeStruct((_B, _N, 1), jnp.float32),
      ],
  )(features, wc, bc, W_box, bb, priors)

  amax_pad = jnp.full((_B, _NPAD), _NEG, jnp.float32)
  amax_pad = amax_pad.at[:, :_N].set(amax[:, :, 0])
  scores_flat = scores.reshape(_B * _N, 128)

  mesh = plsc.VectorSubcoreMesh(core_axis_name="c", subcore_axis_name="s")
  sc_call = pl.kernel(
      _sc_body,
      mesh=mesh,
      out_type=[
          jax.ShapeDtypeStruct((_B, 4, 128), jnp.float32),
          jax.ShapeDtypeStruct((_B, 128), jnp.int32),
          jax.ShapeDtypeStruct((_B, 128), jnp.float32),
      ],
      scratch_types=[
          pltpu.VMEM((_NPAD,), jnp.float32),
          pltpu.VMEM((320,), jnp.float32),
          pltpu.VMEM((32,), jnp.float32),
          pltpu.VMEM((128,), jnp.int32),
          pltpu.VMEM((128, 128), jnp.float32),
          pltpu.VMEM((1024,), jnp.float32),
          pltpu.VMEM((64,), jnp.float32),
          pltpu.VMEM((128,), jnp.float32),
          pltpu.VMEM((128,), jnp.int32),
      ] + [pltpu.VMEM((128,), jnp.float32) for _ in range(10)] + [
          pltpu.VMEM((4, 128), jnp.float32),
          pltpu.SemaphoreType.DMA,
      ],
  )
  boxes_o, labels_o, scores_o = sc_call(amax_pad, scores_flat)

  boxes_out = jnp.transpose(boxes_o, (0, 2, 1))[:, :_MAX_CAND, :]
  labels_out = labels_o[:, :_MAX_CAND]
  scores_out = scores_o[:, :_MAX_CAND]
  img_w = jnp.full((_B,), _IMG, dtype=jnp.float32)
  img_h = jnp.full((_B,), _IMG, dtype=jnp.float32)
  return boxes_out, labels_out, scores_out, img_w, img_h


# speedup vs baseline: 1.0153x; 1.0128x over previous
import jax
import jax.numpy as jnp
from jax import lax
from jax.experimental import pallas as pl
from jax.experimental.pallas import tpu as pltpu
from jax.experimental.pallas import tpu_sc as plsc

_B = 8
_N = 5000
_C = 81
_D = 256
_IMG = 512.0
_CENTER_VAR = 0.1
_SIZE_VAR = 0.2
_NMS_THRESH = 0.45
_MAX_CAND = 100
_NEG = -1e30
_NPAD = 5120


def _stage1_body(feat_ref, wc_ref, bc_ref, wb_ref, bb_ref, pri_ref,
                 scores_ref, amax_ref):
  feat = feat_ref[0]
  logits = jnp.dot(feat, wc_ref[...], preferred_element_type=jnp.float32)
  logits = logits + bc_ref[...]
  m = jnp.max(logits, axis=1, keepdims=True)
  p = logits - m
  lse = jnp.log(jnp.sum(jnp.exp(p), axis=1, keepdims=True))
  logp = p - lse
  lane = lax.broadcasted_iota(jnp.int32, (_N, 128), 1)
  valid = jnp.logical_and(lane >= 1, lane <= _C - 1)
  scores = jnp.where(valid, logp, _NEG)
  amax_ref[0] = jnp.max(scores, axis=1, keepdims=True)

  loc = jnp.dot(feat, wb_ref[...], preferred_element_type=jnp.float32)
  loc = loc + bb_ref[...]
  pri = pri_ref[...]
  pw = pri[:, 2:3]
  ph = pri[:, 3:4]
  cx = loc[:, 0:1] * _CENTER_VAR * pw + pri[:, 0:1]
  cy = loc[:, 1:2] * _CENTER_VAR * ph + pri[:, 1:2]
  w = jnp.exp(loc[:, 2:3] * _SIZE_VAR) * pw
  h = jnp.exp(loc[:, 3:4] * _SIZE_VAR) * ph
  boxmat = jnp.concatenate(
      [jnp.zeros((_N, 96), jnp.float32),
       cx - w * 0.5, cy - h * 0.5, cx + w * 0.5, cy + h * 0.5,
       jnp.zeros((_N, 28), jnp.float32)], axis=1)
  emb = jnp.logical_and(lane >= 96, lane < 100)
  scores_ref[0] = jnp.where(emb, boxmat, scores)


_GDN = lax.GatherDimensionNumbers(
    offset_dims=(), collapsed_slice_dims=(0,), start_index_map=(0,))


def _shuf(v, idx):
  return lax.gather(v, idx[:, None], dimension_numbers=_GDN,
                    slice_sizes=(1,), mode=lax.GatherScatterMode.PROMISE_IN_BOUNDS)


def _vmax(v, lane16):
  for s in (8, 4, 2, 1):
    v = jnp.maximum(v, _shuf(v, lane16 ^ s))
  return v[0]


def _lanebcast(v, lane16, i):
  return _shuf(v, lane16 * 0 + i)


def _vmin_s(c, lane16):
  for s in (8, 4, 2, 1):
    c = jnp.minimum(c, _shuf(c, lane16 ^ s))
  return c[0]


def _ffs(mask, lane16):
  c = jnp.where(mask, lane16, jnp.int32(1 << 20))
  for s in (8, 4, 2, 1):
    c = jnp.minimum(c, _shuf(c, lane16 ^ s))
  return c[0]


def _popcnt(mask, lane16):
  c = jnp.where(mask, jnp.int32(1), jnp.int32(0))
  for s in (8, 4, 2, 1):
    c = c + _shuf(c, lane16 ^ s)
  return c[0]


def _sc_body(amax_hbm, scores_hbm,
             boxes_o, labels_o, scores_o,
             s_amax, s_l1, s_l2, s_sorted, s_rows,
             s_p1, s_p2, cnd_s, cnd_c,
             rx1, ry1, rx2, ry2, cx1, cy1, cx2, cy2, s_area, s_supp,
             s_ob, sem):
  cidx = lax.axis_index("c")
  sidx = lax.axis_index("s")

  @pl.when(jnp.logical_and(cidx == 0, sidx < _B))
  def _():
    img = sidx
    base = img * _N
    lane16 = lax.iota(jnp.int32, 16)
    negv = jnp.full((16,), _NEG, jnp.float32)

    pltpu.sync_copy(amax_hbm.at[img], s_amax)

    def build_l1(o, _):
      acc = negv
      for t in range(16):
        val = _vmax(s_amax[pl.ds(o * 256 + t * 16, 16)], lane16)
        acc = jnp.where(lane16 == t, val, acc)
      s_l1[pl.ds(o * 16, 16)] = acc
      return 0
    lax.fori_loop(0, 20, build_l1, 0)

    for o in range(2):
      acc = negv
      for t in range(16 if o == 0 else 4):
        val = _vmax(s_l1[pl.ds((o * 16 + t) * 16, 16)], lane16)
        acc = jnp.where(lane16 == t, val, acc)
      s_l2[pl.ds(o * 16, 16)] = acc

    def extract_a(k, _):
      va = s_l2[pl.ds(0, 16)]
      vb = s_l2[pl.ds(16, 16)]
      v = _vmax(jnp.maximum(va, vb), lane16)
      big = jnp.int32(1 << 20)
      ca = jnp.where(va == v, lane16, big)
      cb = jnp.where(vb == v, lane16 + 16, big)
      j2 = _vmin_s(jnp.minimum(ca, cb), lane16)
      lv = s_l1[pl.ds(j2 * 16, 16)]
      j1 = j2 * 16 + _ffs(lv == v, lane16)
      av = s_amax[pl.ds(j1 * 16, 16)]
      alane = _ffs(av == v, lane16)
      s_amax[pl.ds(j1 * 16, 16)] = jnp.where(lane16 == alane, _NEG, av)
      nv1 = _vmax(s_amax[pl.ds(j1 * 16, 16)], lane16)
      o1 = (j1 // 16) * 16
      s_l1[pl.ds(o1, 16)] = jnp.where(lane16 == j1 % 16, nv1,
                                      s_l1[pl.ds(o1, 16)])
      nv2 = _vmax(s_l1[pl.ds(j2 * 16, 16)], lane16)
      o2 = (j2 // 16) * 16
      s_l2[pl.ds(o2, 16)] = jnp.where(lane16 == j2 % 16, nv2,
                                      s_l2[pl.ds(o2, 16)])
      return 0
    lax.fori_loop(0, _MAX_CAND, extract_a, 0)

    def init_sorted(j, _):
      s_sorted[pl.ds(j * 16, 16)] = lane16 + (base + j * 16)
      return 0
    lax.fori_loop(0, 8, init_sorted, 0)

    def compact(j, cnt):
      av = s_amax[pl.ds(j * 16, 16)]
      flat = lane16 + j * 16
      msk = jnp.logical_and(av == _NEG, flat < _N)
      mvec = jnp.where(msk, jnp.int32(1), jnp.int32(0))
      n = _popcnt(msk, lane16)

      def inner(_, st):
        mcur, c = st
        lanei = _ffs(mcur == 1, lane16)
        anchor = j * 16 + lanei + base
        co = (c // 16) * 16
        s_sorted[pl.ds(co, 16)] = jnp.where(lane16 == c % 16, anchor,
                                            s_sorted[pl.ds(co, 16)])
        mnext = jnp.where(lane16 == lanei, jnp.int32(0), mcur)
        return (mnext, c + 1)

      st = lax.fori_loop(0, n, inner, (mvec, cnt))
      return st[1]
    lax.fori_loop(0, 313, compact, jnp.int32(0))

    pltpu.async_copy(scores_hbm.at[s_sorted], s_rows, sem).wait()

    def build_p1(o, _):
      acc = negv
      for t in range(16):
        r = o * 2 + t // 8
        c0 = (t % 8) * 16
        x = s_rows[r, pl.ds(c0, 16)]
        if t % 8 == 6:
          val = _NEG
        else:
          val = jnp.where(r < _MAX_CAND, _vmax(x, lane16), _NEG)
        acc = jnp.where(lane16 == t, val, acc)
      s_p1[pl.ds(o * 16, 16)] = acc
      return 0
    lax.fori_loop(0, 64, build_p1, 0)

    def build_p2(o, _):
      acc = negv
      for t in range(16):
        val = _vmax(s_p1[pl.ds((o * 16 + t) * 16, 16)], lane16)
        acc = jnp.where(lane16 == t, val, acc)
      s_p2[pl.ds(o * 16, 16)] = acc
      return 0
    lax.fori_loop(0, 4, build_p2, 0)

    def init_cand(j, _):
      z = jnp.zeros((16,), jnp.float32)
      cnd_s[pl.ds(j * 16, 16)] = z
      cnd_c[pl.ds(j * 16, 16)] = jnp.zeros((16,), jnp.int32)
      rx1[pl.ds(j * 16, 16)] = z
      ry1[pl.ds(j * 16, 16)] = z
      rx2[pl.ds(j * 16, 16)] = z
      ry2[pl.ds(j * 16, 16)] = z
      cx1[pl.ds(j * 16, 16)] = z
      cy1[pl.ds(j * 16, 16)] = z
      cx2[pl.ds(j * 16, 16)] = z
      cy2[pl.ds(j * 16, 16)] = z
      s_supp[pl.ds(j * 16, 16)] = z
      return 0
    lax.fori_loop(0, 8, init_cand, 0)

    def extract_p(k, _):
      p20 = s_p2[pl.ds(0, 16)]
      p21 = s_p2[pl.ds(16, 16)]
      p22 = s_p2[pl.ds(32, 16)]
      p23 = s_p2[pl.ds(48, 16)]
      v = _vmax(jnp.maximum(jnp.maximum(p20, p21),
                            jnp.maximum(p22, p23)), lane16)
      big = jnp.int32(1 << 20)
      c0_ = jnp.where(p20 == v, lane16, big)
      c1_ = jnp.where(p21 == v, lane16 + 16, big)
      c2_ = jnp.where(p22 == v, lane16 + 32, big)
      c3_ = jnp.where(p23 == v, lane16 + 48, big)
      t = _vmin_s(jnp.minimum(jnp.minimum(c0_, c1_),
                              jnp.minimum(c2_, c3_)), lane16)
      pv = s_p1[pl.ds(t * 16, 16)]
      j = t * 16 + _ffs(pv == v, lane16)
      r = j // 8
      c0 = (j % 8) * 16
      leaf = s_rows[r, pl.ds(c0, 16)]
      lne = _ffs(leaf == v, lane16)
      c = c0 + lne
      ko = (k // 16) * 16
      ksl = pl.ds(ko, 16)
      klane = lane16 == k % 16
      sc = v
      cnd_s[ksl] = jnp.where(klane, sc, cnd_s[ksl])
      cnd_c[ksl] = jnp.where(klane, c, cnd_c[ksl])
      bv = s_rows[r, pl.ds(96, 16)]
      x1 = bv[0] * _IMG
      y1 = bv[1] * _IMG
      x2 = bv[2] * _IMG
      y2 = bv[3] * _IMG
      off = c.astype(jnp.float32) * (_IMG * 4.0)
      rx1[ksl] = jnp.where(klane, x1, rx1[ksl])
      ry1[ksl] = jnp.where(klane, y1, ry1[ksl])
      rx2[ksl] = jnp.where(klane, x2, rx2[ksl])
      ry2[ksl] = jnp.where(klane, y2, ry2[ksl])
      cx1[ksl] = jnp.where(klane, x1 + off, cx1[ksl])
      cy1[ksl] = jnp.where(klane, y1 + off, cy1[ksl])
      cx2[ksl] = jnp.where(klane, x2 + off, cx2[ksl])
      cy2[ksl] = jnp.where(klane, y2 + off, cy2[ksl])
      s_rows[r, pl.ds(c0, 16)] = jnp.where(lane16 == lne, _NEG, leaf)
      nv1 = _vmax(s_rows[r, pl.ds(c0, 16)], lane16)
      o1 = (j // 16) * 16
      s_p1[pl.ds(o1, 16)] = jnp.where(lane16 == j % 16, nv1,
                                      s_p1[pl.ds(o1, 16)])
      t2 = j // 16
      nv2 = _vmax(s_p1[pl.ds(t2 * 16, 16)], lane16)
      o2 = (t2 // 16) * 16
      s_p2[pl.ds(o2, 16)] = jnp.where(lane16 == t2 % 16, nv2,
                                      s_p2[pl.ds(o2, 16)])
      return 0
    lax.fori_loop(0, _MAX_CAND, extract_p, 0)

    def build_area(j, _):
      w = jnp.maximum(cx2[pl.ds(j * 16, 16)] - cx1[pl.ds(j * 16, 16)], 0.0)
      h = jnp.maximum(cy2[pl.ds(j * 16, 16)] - cy1[pl.ds(j * 16, 16)], 0.0)
      s_area[pl.ds(j * 16, 16)] = w * h
      return 0
    lax.fori_loop(0, 8, build_area, 0)

    def nms_step(i, _):
      io = (i // 16) * 16
      isl = pl.ds(io, 16)
      ilane = lane16 == i % 16

      def pick(ref):
        return _lanebcast(ref[isl], lane16, i % 16)

      live = jnp.where(pick(s_supp) == 0.0, 1.0, 0.0)
      x1i = pick(cx1)
      y1i = pick(cy1)
      x2i = pick(cx2)
      y2i = pick(cy2)
      ai = pick(s_area)
      for j in range(8):
        sl = pl.ds(j * 16, 16)
        ltx = jnp.maximum(x1i, cx1[sl])
        lty = jnp.maximum(y1i, cy1[sl])
        rbx = jnp.minimum(x2i, cx2[sl])
        rby = jnp.minimum(y2i, cy2[sl])
        inter = jnp.maximum(rbx - ltx, 0.0) * jnp.maximum(rby - lty, 0.0)
        union = ai + s_area[sl] - inter
        iou = inter / jnp.maximum(union, 1e-9)
        glane = lane16 + j * 16
        hit = jnp.logical_and(iou > _NMS_THRESH, glane > i)
        s_supp[sl] = jnp.maximum(s_supp[sl],
                                 jnp.where(hit, live, 0.0))
      return 0
    lax.fori_loop(0, _MAX_CAND, nms_step, 0)

    def mask_vec(j, _):
      sl = pl.ds(j * 16, 16)
      kp = s_supp[sl] == 0.0
      f = jnp.where(kp, 1.0, 0.0)
      cnd_s[sl] = jnp.where(kp, cnd_s[sl], -1e4)
      cnd_c[sl] = jnp.where(kp, cnd_c[sl], -1)
      s_ob[0, sl] = rx1[sl] * f
      s_ob[1, sl] = ry1[sl] * f
      s_ob[2, sl] = rx2[sl] * f
      s_ob[3, sl] = ry2[sl] * f
      return 0
    lax.fori_loop(0, 8, mask_vec, 0)

    pltpu.sync_copy(s_ob, boxes_o.at[img])
    pltpu.sync_copy(cnd_c, labels_o.at[img])
    pltpu.sync_copy(cnd_s, scores_o.at[img])


@jax.jit
def kernel(features, W_cls, b_cls, W_box, b_box, priors):
  wc = jnp.zeros((_D, 128), jnp.float32).at[:, :_C].set(W_cls)
  bc = jnp.full((1, 128), _NEG, jnp.float32).at[0, :_C].set(b_cls)
  bb = b_box.reshape(1, 4)

  scores, amax = pl.pallas_call(
      _stage1_body,
      grid=(_B,),
      in_specs=[
          pl.BlockSpec((1, _N, _D), lambda i: (i, 0, 0)),
          pl.BlockSpec((_D, 128), lambda i: (0, 0)),
          pl.BlockSpec((1, 128), lambda i: (0, 0)),
          pl.BlockSpec((_D, 4), lambda i: (0, 0)),
          pl.BlockSpec((1, 4), lambda i: (0, 0)),
          pl.BlockSpec((_N, 4), lambda i: (0, 0)),
      ],
      out_specs=[
          pl.BlockSpec((1, _N, 128), lambda i: (i, 0, 0)),
          pl.BlockSpec((1, _N, 1), lambda i: (i, 0, 0)),
      ],
      out_shape=[
          jax.ShapeDtypeStruct((_B, _N, 128), jnp.float32),
          jax.ShapeDtypeStruct((_B, _N, 1), jnp.float32),
      ],
  )(features, wc, bc, W_box, bb, priors)

  amax_pad = jnp.full((_B, _NPAD), _NEG, jnp.float32)
  amax_pad = amax_pad.at[:, :_N].set(amax[:, :, 0])
  scores_flat = scores.reshape(_B * _N, 128)

  mesh = plsc.VectorSubcoreMesh(core_axis_name="c", subcore_axis_name="s")
  sc_call = pl.kernel(
      _sc_body,
      mesh=mesh,
      out_type=[
          jax.ShapeDtypeStruct((_B, 4, 128), jnp.float32),
          jax.ShapeDtypeStruct((_B, 128), jnp.int32),
          jax.ShapeDtypeStruct((_B, 128), jnp.float32),
      ],
      scratch_types=[
          pltpu.VMEM((_NPAD,), jnp.float32),
          pltpu.VMEM((320,), jnp.float32),
          pltpu.VMEM((32,), jnp.float32),
          pltpu.VMEM((128,), jnp.int32),
          pltpu.VMEM((128, 128), jnp.float32),
          pltpu.VMEM((1024,), jnp.float32),
          pltpu.VMEM((64,), jnp.float32),
          pltpu.VMEM((128,), jnp.float32),
          pltpu.VMEM((128,), jnp.int32),
      ] + [pltpu.VMEM((128,), jnp.float32) for _ in range(10)] + [
          pltpu.VMEM((4, 128), jnp.float32),
          pltpu.SemaphoreType.DMA,
      ],
  )
  boxes_o, labels_o, scores_o = sc_call(amax_pad, scores_flat)

  boxes_out = jnp.transpose(boxes_o, (0, 2, 1))[:, :_MAX_CAND, :]
  labels_out = labels_o[:, :_MAX_CAND]
  scores_out = scores_o[:, :_MAX_CAND]
  img_w = jnp.full((_B,), _IMG, dtype=jnp.float32)
  img_h = jnp.full((_B,), _IMG, dtype=jnp.float32)
  return boxes_out, labels_out, scores_out, img_w, img_h
